# baseline (speedup 1.0000x reference)
import jax
import jax.numpy as jnp
from jax.experimental import pallas as pl
from jax.experimental.pallas import tpu as pltpu


def _ew_kernel(bias_ref, scores_hbm, masks_hbm, out_hbm,
               scores_v, masks_v, out_v, s_sem, m_sem, o_sem):
    cp_s = pltpu.make_async_copy(scores_hbm, scores_v, s_sem)
    cp_m = pltpu.make_async_copy(masks_hbm, masks_v, m_sem)
    cp_s.start()
    cp_m.start()
    b = bias_ref[()]
    cp_s.wait()
    cp_m.wait()
    out_v[...] = (scores_v[...] + b) * masks_v[...].astype(jnp.float32)
    cp_o = pltpu.make_async_copy(out_v, out_hbm, o_sem)
    cp_o.start()
    cp_o.wait()


def kernel(sent_group_scores, sel_sent_emb, sel_sent_masks, group_embs, candi_sent_masks, bias):
    del sel_sent_emb, sel_sent_masks, group_embs
    B, S = sent_group_scores.shape
    return pl.pallas_call(
        _ew_kernel,
        in_specs=[
            pl.BlockSpec(memory_space=pltpu.SMEM),
            pl.BlockSpec(memory_space=pl.ANY),
            pl.BlockSpec(memory_space=pl.ANY),
        ],
        out_specs=pl.BlockSpec(memory_space=pl.ANY),
        out_shape=jax.ShapeDtypeStruct((B, S), jnp.float32),
        scratch_shapes=[
            pltpu.VMEM((B, S), jnp.float32),
            pltpu.VMEM((B, S), jnp.int32),
            pltpu.VMEM((B, S), jnp.float32),
            pltpu.SemaphoreType.DMA,
            pltpu.SemaphoreType.DMA,
            pltpu.SemaphoreType.DMA,
        ],
    )(bias, sent_group_scores, candi_sent_masks)
